# baseline (speedup 1.0000x reference)
import functools

import jax
import jax.numpy as jnp
from jax import lax
from jax.experimental import pallas as pl
from jax.experimental.pallas import tpu as pltpu
from jax.experimental.pallas import tpu_sc as plsc

_VOCAB = 100000
_VPAD = 102400
_D = 64
_BATCH = 4096
_SEQ = 200

_NC = 2
_NS = 16
_NW = _NC * _NS

_GRID = 4
_VLANES = _VPAD // _GRID
_GPB = (_BATCH // 128) // _GRID


def _tc_body(xt_ref, w_ref, b_ref, idst_ref, proj_ref, ids_ref):
    wb = w_ref[...]
    scale = jnp.float32(1.0 / _SEQ)
    bias = b_ref[0, 0] * scale
    for t in range(_VLANES // 1024):
        rows = []
        for gg in range(8):
            g = t * 8 + gg
            blk = xt_ref[:, g * 128:(g + 1) * 128]
            rows.append(jnp.sum(blk * wb, axis=0, keepdims=True) * scale)
        proj_ref[pl.ds(t * 8, 8), :] = jnp.concatenate(rows, axis=0) + bias
    for gg in range(_GPB):
        ids_ref[gg, :, :] = idst_ref[:, gg * 128:(gg + 1) * 128]


def _tc_stage(embed_table, head_w, head_b, input_ids):
    return pl.pallas_call(
        _tc_body,
        grid=(_GRID,),
        in_specs=[
            pl.BlockSpec((_D, _VLANES), lambda i: (0, i)),
            pl.BlockSpec((_D, 1), lambda i: (0, 0)),
            pl.BlockSpec((1, 1), lambda i: (0, 0)),
            pl.BlockSpec((_SEQ, _GPB * 128), lambda i: (0, i)),
        ],
        out_specs=[
            pl.BlockSpec((_VLANES // 128, 128), lambda i: (i, 0)),
            pl.BlockSpec((_GPB, _SEQ, 128), lambda i: (i, 0, 0)),
        ],
        out_shape=[
            jax.ShapeDtypeStruct((_VPAD // 128, 128), jnp.float32),
            jax.ShapeDtypeStruct((_NW, _SEQ, 128), jnp.int32),
        ],
    )(embed_table.T, head_w, head_b.reshape(1, 1),
      input_ids.astype(jnp.int32).T)


_ROWS_PER_W = _BATCH // _NW
_UNROLL = 8


_NJ = _ROWS_PER_W // 16


def _sc_body(proj_hbm, ids_hbm, out_hbm, proj_v, ids_v, out_v):
    wid = lax.axis_index("s") * _NC + lax.axis_index("c")
    pltpu.sync_copy(proj_hbm, proj_v)
    pltpu.sync_copy(ids_hbm.at[wid], ids_v)
    lane = lax.iota(jnp.int32, 16)
    rowbase = [lane + (j * 16) for j in range(_NJ)]

    def body(s0, carry):
        sv, *accs = carry
        for _ in range(_UNROLL):
            for j in range(_NJ):
                idx = plsc.load_gather(ids_v, [sv, rowbase[j]])
                accs[j] = accs[j] + plsc.load_gather(proj_v, [idx])
            sv = sv + 1
        return (sv, *accs)

    zero = jnp.zeros((16,), jnp.float32)
    carry = lax.fori_loop(
        0, _SEQ // _UNROLL, body,
        (jnp.zeros((16,), jnp.int32), *([zero] * _NJ)))
    for j in range(_NJ):
        out_v[pl.ds(j * 16, 16)] = carry[1 + j]
    pltpu.sync_copy(out_v, out_hbm.at[pl.ds(wid * _ROWS_PER_W, _ROWS_PER_W)])


def _gather_sum(proj1d, ids2d):
    mesh = plsc.VectorSubcoreMesh(core_axis_name="c", subcore_axis_name="s")
    run = functools.partial(
        pl.kernel,
        mesh=mesh,
        compiler_params=pltpu.CompilerParams(needs_layout_passes=False),
        out_type=jax.ShapeDtypeStruct((_BATCH,), jnp.float32),
        scratch_types=[
            pltpu.VMEM((_VPAD,), jnp.float32),
            pltpu.VMEM((_SEQ, 128), jnp.int32),
            pltpu.VMEM((_ROWS_PER_W,), jnp.float32),
        ],
    )(_sc_body)
    return run(proj1d, ids2d)


def kernel(input_ids, embed_table, head_w, head_b):
    proj2d, ids_packed = _tc_stage(embed_table, head_w, head_b, input_ids)
    return _gather_sum(proj2d.reshape(_VPAD), ids_packed)


# speedup vs baseline: 1.0233x; 1.0233x over previous
import functools

import jax
import jax.numpy as jnp
from jax import lax
from jax.experimental import pallas as pl
from jax.experimental.pallas import tpu as pltpu
from jax.experimental.pallas import tpu_sc as plsc

_VOCAB = 100000
_VPAD = 102400
_D = 64
_BATCH = 4096
_SEQ = 200

_NC = 2
_NS = 16
_NW = _NC * _NS

_GRID = 4
_VLANES = _VPAD // _GRID
_GPB = (_BATCH // 128) // _GRID


def _tc_body(xt_ref, w_ref, b_ref, idst_ref, proj_ref, ids_ref):
    wb = w_ref[...]
    scale = jnp.float32(1.0 / _SEQ)
    bias = b_ref[0, 0] * scale
    for t in range(_VLANES // 1024):
        rows = []
        for gg in range(8):
            g = t * 8 + gg
            blk = xt_ref[:, g * 128:(g + 1) * 128]
            rows.append(jnp.sum(blk * wb, axis=0, keepdims=True) * scale)
        proj_ref[pl.ds(t * 8, 8), :] = jnp.concatenate(rows, axis=0) + bias
    for gg in range(_GPB):
        ids_ref[gg, :, :] = idst_ref[:, gg * 128:(gg + 1) * 128]


def _tc_stage(embed_table, head_w, head_b, input_ids):
    return pl.pallas_call(
        _tc_body,
        grid=(_GRID,),
        in_specs=[
            pl.BlockSpec((_D, _VLANES), lambda i: (0, i)),
            pl.BlockSpec((_D, 1), lambda i: (0, 0)),
            pl.BlockSpec((1, 1), lambda i: (0, 0)),
            pl.BlockSpec((_SEQ, _GPB * 128), lambda i: (0, i)),
        ],
        out_specs=[
            pl.BlockSpec((_VLANES // 128, 128), lambda i: (i, 0)),
            pl.BlockSpec((_GPB, _SEQ, 128), lambda i: (i, 0, 0)),
        ],
        out_shape=[
            jax.ShapeDtypeStruct((_VPAD // 128, 128), jnp.float32),
            jax.ShapeDtypeStruct((_NW, _SEQ, 128), jnp.int32),
        ],
    )(embed_table.T, head_w, head_b.reshape(1, 1),
      input_ids.astype(jnp.int32).T)


_ROWS_PER_W = _BATCH // _NW
_UNROLL = 8


def _sc_body(proj_hbm, ids_hbm, out_hbm, proj_v, ids_v, out_v):
    wid = lax.axis_index("s") * _NC + lax.axis_index("c")
    pltpu.sync_copy(proj_hbm, proj_v)
    pltpu.sync_copy(ids_hbm.at[wid], ids_v)
    lane = lax.iota(jnp.int32, 16)
    for j in range(_ROWS_PER_W // 16):
        row = lane + (j * 16)

        def body(s0, acc, row=row):
            for u in range(_UNROLL):
                s = jnp.zeros((16,), jnp.int32) + (s0 * _UNROLL + u)
                idx = plsc.load_gather(ids_v, [s, row])
                acc = acc + plsc.load_gather(proj_v, [idx >> 7, idx & 127])
            return acc

        acc = lax.fori_loop(0, _SEQ // _UNROLL, body,
                            jnp.zeros((16,), jnp.float32))
        out_v[pl.ds(j * 16, 16)] = acc
    pltpu.sync_copy(out_v, out_hbm.at[pl.ds(wid * _ROWS_PER_W, _ROWS_PER_W)])


def _gather_sum(proj2d, ids_packed):
    mesh = plsc.VectorSubcoreMesh(core_axis_name="c", subcore_axis_name="s")
    run = functools.partial(
        pl.kernel,
        mesh=mesh,
        compiler_params=pltpu.CompilerParams(needs_layout_passes=False),
        out_type=jax.ShapeDtypeStruct((_BATCH,), jnp.float32),
        scratch_types=[
            pltpu.VMEM((_VPAD // 128, 128), jnp.float32),
            pltpu.VMEM((_SEQ, 128), jnp.int32),
            pltpu.VMEM((_ROWS_PER_W,), jnp.float32),
        ],
    )(_sc_body)
    return run(proj2d, ids_packed)


def kernel(input_ids, embed_table, head_w, head_b):
    proj2d, ids_packed = _tc_stage(embed_table, head_w, head_b, input_ids)
    return _gather_sum(proj2d, ids_packed)
